# Initial kernel scaffold; baseline (speedup 1.0000x reference)
import jax
import jax.numpy as jnp
from jax.experimental import pallas as pl


def kernel(x, emb, W1, b1, W2, b2):
    raise NotImplementedError("write your pallas kernel here")


# speedup vs baseline: 1.3052x; 1.3052x over previous
import functools

import jax
import jax.numpy as jnp
from jax import lax
from jax.experimental import pallas as pl
from jax.experimental.pallas import tpu as pltpu
from jax.experimental.pallas import tpu_sc as plsc

B, S, D = 4096, 50, 300
DP = 304
H, C = 256, 5
NC, NS, L = 2, 16, 16
NW = NC * NS
BPW = B // NW

WIN = list(range(0, D - L + 1, L))
if WIN[-1] != D - L:
    WIN.append(D - L)


def _pool_body(x_hbm, emb_hbm, out_hbm, idx_v, rows0, rows1, acc_v, sem0, sem1):
    wid = lax.axis_index("s") * NC + lax.axis_index("c")
    base = wid * BPW
    pltpu.sync_copy(x_hbm.at[pl.ds(base, BPW)], idx_v)

    def gather(b, buf, sem):
        pltpu.async_copy(emb_hbm.at[idx_v.at[b]], buf, sem)

    def wait(b, buf, sem):
        pltpu.make_async_copy(emb_hbm.at[idx_v.at[b]], buf, sem).wait()

    def compute(b, buf):
        acc = tuple(buf[0, pl.ds(o, L)] for o in WIN)

        def sbody(s, acc):
            return tuple(a + buf[s, pl.ds(o, L)] for a, o in zip(acc, WIN))

        acc = lax.fori_loop(1, S, sbody, acc)
        inv = jnp.float32(1.0 / S)
        for a, o in zip(acc, WIN):
            acc_v[b, pl.ds(o, L)] = a * inv

    gather(0, rows0, sem0)
    gather(1, rows1, sem1)

    def loop_body(g, carry):
        b = g * 2
        wait(b, rows0, sem0)
        compute(b, rows0)

        @pl.when(b + 2 < BPW)
        def _():
            gather(b + 2, rows0, sem0)

        wait(b + 1, rows1, sem1)
        compute(b + 1, rows1)

        @pl.when(b + 3 < BPW)
        def _():
            gather(b + 3, rows1, sem1)

        return carry

    lax.fori_loop(0, BPW // 2, loop_body, 0)
    pltpu.sync_copy(acc_v, out_hbm.at[pl.ds(base, BPW)])


_pool = functools.partial(
    pl.kernel,
    out_type=jax.ShapeDtypeStruct((B, D), jnp.float32),
    mesh=plsc.VectorSubcoreMesh(core_axis_name="c", subcore_axis_name="s"),
    scratch_types=[
        pltpu.VMEM((BPW, S), jnp.int32),
        pltpu.VMEM((S, DP), jnp.float32),
        pltpu.VMEM((S, DP), jnp.float32),
        pltpu.VMEM((BPW, D), jnp.float32),
        pltpu.SemaphoreType.DMA,
        pltpu.SemaphoreType.DMA,
    ],
    compiler_params=pltpu.CompilerParams(use_tc_tiling_on_sc=False),
)(_pool_body)


def _mlp_body(h_ref, w1t_ref, b1_ref, w2t_ref, b2_ref, o_ref):
    h = h_ref[...]
    z = jnp.dot(h, w1t_ref[...], preferred_element_type=jnp.float32)
    z = jnp.maximum(z + b1_ref[...], 0.0)
    o_ref[...] = jnp.dot(z, w2t_ref[...], preferred_element_type=jnp.float32) + b2_ref[...]


def kernel(x, emb, W1, b1, W2, b2):
    emb_p = jnp.pad(emb, ((0, 0), (0, DP - D)))
    pooled = _pool(x.astype(jnp.int32), emb_p)
    return pl.pallas_call(
        _mlp_body,
        out_shape=jax.ShapeDtypeStruct((B, C), jnp.float32),
    )(pooled, W1.T, b1.reshape(1, H), W2.T, b2.reshape(1, C))


# speedup vs baseline: 1.3403x; 1.0269x over previous
import functools

import jax
import jax.numpy as jnp
from jax import lax
from jax.experimental import pallas as pl
from jax.experimental.pallas import tpu as pltpu
from jax.experimental.pallas import tpu_sc as plsc

B, S, D = 4096, 50, 300
V = 100000
H, C = 256, 5
NC, NS, L = 2, 16, 16
NW = NC * NS
BPW = B // NW
NIDX = BPW * S
GMAX = (V * D) // 16 - 20

WIN = list(range(0, D - L + 1, L))
if WIN[-1] != D - L:
    WIN.append(D - L)


def _pool_body(x_hbm, embf_hbm, out_hbm, idx_f, buf0, buf1, acc_v, sem0, sem1):
    wid = lax.axis_index("s") * NC + lax.axis_index("c")
    base = wid * BPW
    pltpu.sync_copy(x_hbm.at[pl.ds(base * S, NIDX)], idx_f.at[pl.ds(0, NIDX)])

    def rd(b, t):
        return idx_f[pl.ds(b * S + t, L)][0]

    def issue(b, buf, sem):
        def tbody(t, carry):
            r = rd(b, t)
            g0 = jnp.minimum((r * 75) >> 2, GMAX)
            pltpu.async_copy(embf_hbm.at[pl.ds(g0 * 16, 320)],
                             buf.at[pl.ds(t * 320, 320)], sem)
            return carry

        lax.fori_loop(0, S, tbody, 0)

    def wait_all(buf, sem):
        pltpu.make_async_copy(embf_hbm.at[pl.ds(0, S * 320)], buf, sem).wait()

    def accumulate(b, buf):
        def tbody(t, acc):
            r = rd(b, t)
            g0 = jnp.minimum((r * 75) >> 2, GMAX)
            start = t * 320 + r * 300 - g0 * 16
            return tuple(a + buf[pl.ds(start + o, L)] for a, o in zip(acc, WIN))

        acc = lax.fori_loop(0, S, tbody,
                            tuple(jnp.zeros((L,), jnp.float32) for _ in WIN))
        inv = jnp.float32(1.0 / S)
        for a, o in zip(acc, WIN):
            acc_v[b, pl.ds(o, L)] = a * inv

    issue(0, buf0, sem0)

    def loop_body(i, carry):
        b = i * 2
        issue(b + 1, buf1, sem1)
        wait_all(buf0, sem0)
        accumulate(b, buf0)

        @pl.when(b + 2 < BPW)
        def _():
            issue(b + 2, buf0, sem0)

        wait_all(buf1, sem1)
        accumulate(b + 1, buf1)
        return carry

    lax.fori_loop(0, BPW // 2, loop_body, 0)
    pltpu.sync_copy(acc_v, out_hbm.at[pl.ds(base, BPW)])


_pool = functools.partial(
    pl.kernel,
    out_type=jax.ShapeDtypeStruct((B, D), jnp.float32),
    mesh=plsc.VectorSubcoreMesh(core_axis_name="c", subcore_axis_name="s"),
    scratch_types=[
        pltpu.VMEM((NIDX + L,), jnp.int32),
        pltpu.VMEM((S * 320,), jnp.float32),
        pltpu.VMEM((S * 320,), jnp.float32),
        pltpu.VMEM((BPW, D), jnp.float32),
        pltpu.SemaphoreType.DMA,
        pltpu.SemaphoreType.DMA,
    ],
    compiler_params=pltpu.CompilerParams(use_tc_tiling_on_sc=False),
)(_pool_body)


def _mlp_body(h_ref, w1t_ref, b1_ref, w2t_ref, b2_ref, o_ref):
    h = h_ref[...]
    z = jnp.dot(h, w1t_ref[...], preferred_element_type=jnp.float32)
    z = jnp.maximum(z + b1_ref[...], 0.0)
    o_ref[...] = jnp.dot(z, w2t_ref[...], preferred_element_type=jnp.float32) + b2_ref[...]


def kernel(x, emb, W1, b1, W2, b2):
    pooled = _pool(x.astype(jnp.int32).reshape(-1), emb.reshape(-1))
    return pl.pallas_call(
        _mlp_body,
        out_shape=jax.ShapeDtypeStruct((B, C), jnp.float32),
    )(pooled, W1.T, b1.reshape(1, H), W2.T, b2.reshape(1, C))


# speedup vs baseline: 2.7140x; 2.0250x over previous
import functools

import jax
import jax.numpy as jnp
from jax import lax
from jax.experimental import pallas as pl
from jax.experimental.pallas import tpu as pltpu
from jax.experimental.pallas import tpu_sc as plsc

B, S, D = 4096, 50, 300
V = 100000
H, C = 256, 5
DA = 256
DB = 48
NC, NS, L = 2, 16, 16
NW = NC * NS
BPW = B // NW
NIDX = BPW * S

WIN_A = list(range(0, DA, L))
WIN_B = [0, 16, 28]


def _pool_a_body(x_hbm, emb_hbm, out_hbm, idx_f, idx_b0, idx_b1,
                 buf0, buf1, acc_v, sem0, sem1):
    wid = lax.axis_index("s") * NC + lax.axis_index("c")
    base = wid * BPW
    pltpu.sync_copy(x_hbm.at[pl.ds(base * S, NIDX)], idx_f.at[pl.ds(0, NIDX)])

    def issue(b, idx_b, buf, sem):
        for o in (0, 16, 32, 34):
            idx_b[pl.ds(o, L)] = idx_f[pl.ds(b * S + o, L)]
        pltpu.async_copy(emb_hbm.at[idx_b, pl.ds(0, 128)],
                         buf.at[:, pl.ds(0, 128)], sem)
        pltpu.async_copy(emb_hbm.at[idx_b, pl.ds(128, 128)],
                         buf.at[:, pl.ds(128, 128)], sem)

    def wait_all(idx_b, buf, sem):
        pltpu.make_async_copy(emb_hbm.at[idx_b, pl.ds(0, 128)],
                              buf.at[:, pl.ds(0, 128)], sem).wait()
        pltpu.make_async_copy(emb_hbm.at[idx_b, pl.ds(128, 128)],
                              buf.at[:, pl.ds(128, 128)], sem).wait()

    def accumulate(b, buf):
        def tbody(t, acc):
            return tuple(a + buf[t, pl.ds(o, L)] for a, o in zip(acc, WIN_A))

        acc = lax.fori_loop(0, S, tbody,
                            tuple(jnp.zeros((L,), jnp.float32) for _ in WIN_A))
        inv = jnp.float32(1.0 / S)
        for a, o in zip(acc, WIN_A):
            acc_v[b, pl.ds(o, L)] = a * inv

    issue(0, idx_b0, buf0, sem0)
    issue(1, idx_b1, buf1, sem1)

    def loop_body(i, carry):
        b = i * 2
        wait_all(idx_b0, buf0, sem0)
        accumulate(b, buf0)

        @pl.when(b + 2 < BPW)
        def _():
            issue(b + 2, idx_b0, buf0, sem0)

        wait_all(idx_b1, buf1, sem1)
        accumulate(b + 1, buf1)

        @pl.when(b + 3 < BPW)
        def _():
            issue(b + 3, idx_b1, buf1, sem1)

        return carry

    lax.fori_loop(0, BPW // 2, loop_body, 0)
    pltpu.sync_copy(acc_v, out_hbm.at[pl.ds(base, BPW)])


_pool_a = functools.partial(
    pl.kernel,
    out_type=jax.ShapeDtypeStruct((B, DA), jnp.float32),
    mesh=plsc.VectorSubcoreMesh(core_axis_name="c", subcore_axis_name="s"),
    scratch_types=[
        pltpu.VMEM((NIDX + L,), jnp.int32),
        pltpu.VMEM((S,), jnp.int32),
        pltpu.VMEM((S,), jnp.int32),
        pltpu.VMEM((S, DA), jnp.float32),
        pltpu.VMEM((S, DA), jnp.float32),
        pltpu.VMEM((BPW, DA), jnp.float32),
        pltpu.SemaphoreType.DMA,
        pltpu.SemaphoreType.DMA,
    ],
    compiler_params=pltpu.CompilerParams(use_tc_tiling_on_sc=True),
)(_pool_a_body)


def _pool_b_body(x_hbm, c2_hbm, out_hbm, idx_v, buf0, buf1, acc_v, sem0, sem1):
    wid = lax.axis_index("s") * NC + lax.axis_index("c")
    base = wid * BPW
    pltpu.sync_copy(x_hbm.at[pl.ds(base, BPW)], idx_v)

    def issue(b, buf, sem):
        pltpu.async_copy(c2_hbm.at[idx_v.at[b]], buf, sem)

    def wait_all(b, buf, sem):
        pltpu.make_async_copy(c2_hbm.at[idx_v.at[b]], buf, sem).wait()

    def accumulate(b, buf):
        def tbody(t, acc):
            return tuple(a + buf[t, pl.ds(o, L)] for a, o in zip(acc, WIN_B))

        acc = lax.fori_loop(0, S, tbody,
                            tuple(jnp.zeros((L,), jnp.float32) for _ in WIN_B))
        inv = jnp.float32(1.0 / S)
        for a, o in zip(acc, WIN_B):
            acc_v[b, pl.ds(o, L)] = a * inv

    issue(0, buf0, sem0)
    issue(1, buf1, sem1)

    def loop_body(i, carry):
        b = i * 2
        wait_all(b, buf0, sem0)
        accumulate(b, buf0)

        @pl.when(b + 2 < BPW)
        def _():
            issue(b + 2, buf0, sem0)

        wait_all(b + 1, buf1, sem1)
        accumulate(b + 1, buf1)

        @pl.when(b + 3 < BPW)
        def _():
            issue(b + 3, buf1, sem1)

        return carry

    lax.fori_loop(0, BPW // 2, loop_body, 0)
    pltpu.sync_copy(acc_v, out_hbm.at[pl.ds(base, BPW)])


_pool_b = functools.partial(
    pl.kernel,
    out_type=jax.ShapeDtypeStruct((B, DB), jnp.float32),
    mesh=plsc.VectorSubcoreMesh(core_axis_name="c", subcore_axis_name="s"),
    scratch_types=[
        pltpu.VMEM((BPW, S), jnp.int32),
        pltpu.VMEM((S, DB), jnp.float32),
        pltpu.VMEM((S, DB), jnp.float32),
        pltpu.VMEM((BPW, DB), jnp.float32),
        pltpu.SemaphoreType.DMA,
        pltpu.SemaphoreType.DMA,
    ],
    compiler_params=pltpu.CompilerParams(use_tc_tiling_on_sc=False),
)(_pool_b_body)


def _mlp_body(pa_ref, pb_ref, w1a_ref, w1b_ref, b1_ref, w2t_ref, b2_ref, o_ref):
    z = jnp.dot(pa_ref[...], w1a_ref[...], preferred_element_type=jnp.float32)
    z = z + jnp.dot(pb_ref[...], w1b_ref[...], preferred_element_type=jnp.float32)
    z = jnp.maximum(z + b1_ref[...], 0.0)
    o_ref[...] = jnp.dot(z, w2t_ref[...], preferred_element_type=jnp.float32) + b2_ref[...]


def kernel(x, emb, W1, b1, W2, b2):
    xi = x.astype(jnp.int32)
    pa = _pool_a(xi.reshape(-1), emb)
    c2p = jnp.pad(emb[:, DA:], ((0, 0), (0, DB - (D - DA))))
    pb = _pool_b(xi, c2p)
    w1a = W1[:, :DA].T
    w1b = jnp.pad(W1[:, DA:], ((0, 0), (0, DB - (D - DA)))).T
    return pl.pallas_call(
        _mlp_body,
        out_shape=jax.ShapeDtypeStruct((B, C), jnp.float32),
    )(pa, pb, w1a, w1b, b1.reshape(1, H), W2.T, b2.reshape(1, C))


# speedup vs baseline: 3.3055x; 1.2179x over previous
import functools

import jax
import jax.numpy as jnp
from jax import lax
from jax.experimental import pallas as pl
from jax.experimental.pallas import tpu as pltpu
from jax.experimental.pallas import tpu_sc as plsc

B, S, D = 4096, 50, 300
V = 100000
H, C = 256, 5
DP = 384
NC, NS, L = 2, 16, 16
NW = NC * NS
BPW = B // NW
NIDX = BPW * S

WIN = list(range(0, 304, L))
ZWIN = [304, 320, 336, 352, 368]


def _pool_body(x_hbm, emb_hbm, c2_hbm, out_hbm, idx_f, idx_b0, idx_b1,
               buf0, buf1, acc_v, sem0, sem1):
    wid = lax.axis_index("s") * NC + lax.axis_index("c")
    base = wid * BPW
    pltpu.sync_copy(x_hbm.at[pl.ds(base * S, NIDX)], idx_f.at[pl.ds(0, NIDX)])

    def issue(b, idx_b, buf, sem):
        for o in (0, 16, 32, 34):
            idx_b[pl.ds(o, L)] = idx_f[pl.ds(b * S + o, L)]
        pltpu.async_copy(emb_hbm.at[idx_b, pl.ds(0, 128)],
                         buf.at[:, pl.ds(0, 128)], sem)
        pltpu.async_copy(emb_hbm.at[idx_b, pl.ds(128, 128)],
                         buf.at[:, pl.ds(128, 128)], sem)
        pltpu.async_copy(c2_hbm.at[idx_b, pl.ds(0, 128)],
                         buf.at[:, pl.ds(256, 128)], sem)

    def wait_all(idx_b, buf, sem):
        pltpu.make_async_copy(emb_hbm.at[idx_b, pl.ds(0, 128)],
                              buf.at[:, pl.ds(0, 128)], sem).wait()
        pltpu.make_async_copy(emb_hbm.at[idx_b, pl.ds(128, 128)],
                              buf.at[:, pl.ds(128, 128)], sem).wait()
        pltpu.make_async_copy(c2_hbm.at[idx_b, pl.ds(0, 128)],
                              buf.at[:, pl.ds(256, 128)], sem).wait()

    zeros = jnp.zeros((L,), jnp.float32)

    def accumulate(b, buf):
        def tbody(t, acc):
            return tuple(a + buf[t, pl.ds(o, L)] for a, o in zip(acc, WIN))

        acc = lax.fori_loop(0, S, tbody,
                            tuple(jnp.zeros((L,), jnp.float32) for _ in WIN))
        inv = jnp.float32(1.0 / S)
        for a, o in zip(acc, WIN):
            acc_v[b, pl.ds(o, L)] = a * inv
        for o in ZWIN:
            acc_v[b, pl.ds(o, L)] = zeros

    issue(0, idx_b0, buf0, sem0)
    issue(1, idx_b1, buf1, sem1)

    def loop_body(i, carry):
        b = i * 2
        wait_all(idx_b0, buf0, sem0)
        accumulate(b, buf0)

        @pl.when(b + 2 < BPW)
        def _():
            issue(b + 2, idx_b0, buf0, sem0)

        wait_all(idx_b1, buf1, sem1)
        accumulate(b + 1, buf1)

        @pl.when(b + 3 < BPW)
        def _():
            issue(b + 3, idx_b1, buf1, sem1)

        return carry

    lax.fori_loop(0, BPW // 2, loop_body, 0)
    pltpu.sync_copy(acc_v, out_hbm.at[pl.ds(base, BPW)])


_pool = functools.partial(
    pl.kernel,
    out_type=jax.ShapeDtypeStruct((B, DP), jnp.float32),
    mesh=plsc.VectorSubcoreMesh(core_axis_name="c", subcore_axis_name="s"),
    scratch_types=[
        pltpu.VMEM((NIDX + L,), jnp.int32),
        pltpu.VMEM((S,), jnp.int32),
        pltpu.VMEM((S,), jnp.int32),
        pltpu.VMEM((S, DP), jnp.float32),
        pltpu.VMEM((S, DP), jnp.float32),
        pltpu.VMEM((BPW, DP), jnp.float32),
        pltpu.SemaphoreType.DMA,
        pltpu.SemaphoreType.DMA,
    ],
    compiler_params=pltpu.CompilerParams(use_tc_tiling_on_sc=True),
)(_pool_body)


def _mlp_body(p_ref, w1t_ref, b1_ref, w2t_ref, b2_ref, o_ref):
    z = jnp.dot(p_ref[...], w1t_ref[...], preferred_element_type=jnp.float32)
    z = jnp.maximum(z + b1_ref[...], 0.0)
    o_ref[...] = jnp.dot(z, w2t_ref[...], preferred_element_type=jnp.float32) + b2_ref[...]


def kernel(x, emb, W1, b1, W2, b2):
    xi = x.astype(jnp.int32)
    c2p = jnp.pad(emb[:, 256:], ((0, 0), (0, 128 - (D - 256))))
    pooled = _pool(xi.reshape(-1), emb, c2p)
    w1t = jnp.pad(W1.T, ((0, DP - D), (0, 0)))
    return pl.pallas_call(
        _mlp_body,
        out_shape=jax.ShapeDtypeStruct((B, C), jnp.float32),
    )(pooled, w1t, b1.reshape(1, H), W2.T, b2.reshape(1, C))
